# Initial kernel scaffold; baseline (speedup 1.0000x reference)
import functools

import jax
import jax.numpy as jnp
from jax import lax
from jax.experimental import pallas as pl
from jax.experimental.pallas import tpu as pltpu
from jax.experimental.pallas import tpu_sc as plsc


def _make_gather3(B, D):
    info = plsc.get_sparse_core_info()
    NC, NS = info.num_cores, info.num_subcores
    NW = NC * NS
    assert B % (8 * NW) == 0
    bpw = B // NW
    mesh = plsc.VectorSubcoreMesh(core_axis_name="c", subcore_axis_name="s")

    @functools.partial(
        pl.kernel,
        out_type=[jax.ShapeDtypeStruct((B, D), jnp.float32)] * 3,
        mesh=mesh,
        scratch_types=[
            pltpu.VMEM((bpw,), jnp.int32),
            pltpu.VMEM((bpw, D), jnp.float32),
            pltpu.VMEM((bpw,), jnp.int32),
            pltpu.VMEM((bpw, D), jnp.float32),
            pltpu.VMEM((bpw,), jnp.int32),
            pltpu.VMEM((bpw, D), jnp.float32),
            pltpu.SemaphoreType.DMA,
            pltpu.SemaphoreType.DMA,
            pltpu.SemaphoreType.DMA,
        ],
    )
    def gather3(e1h, i1h, e2h, i2h, e3h, i3h, o1, o2, o3,
                iv1, rv1, iv2, rv2, iv3, rv3, s1, s2, s3):
        wid = lax.axis_index("s") * NC + lax.axis_index("c")
        base = wid * bpw
        pltpu.sync_copy(i1h.at[pl.ds(base, bpw)], iv1)
        pltpu.sync_copy(i2h.at[pl.ds(base, bpw)], iv2)
        pltpu.sync_copy(i3h.at[pl.ds(base, bpw)], iv3)
        c1 = pltpu.async_copy(e1h.at[iv1], rv1, s1)
        c2 = pltpu.async_copy(e2h.at[iv2], rv2, s2)
        c3 = pltpu.async_copy(e3h.at[iv3], rv3, s3)
        c1.wait()
        pltpu.sync_copy(rv1, o1.at[pl.ds(base, bpw)])
        c2.wait()
        pltpu.sync_copy(rv2, o2.at[pl.ds(base, bpw)])
        c3.wait()
        pltpu.sync_copy(rv3, o3.at[pl.ds(base, bpw)])

    return gather3


def _tc_body(dx, e1, e2, e3, w1, b1, w2, b2, w3, b3,
             t1x, t1i, tb1, t2, tb2, t3r, tb3, out):
    x = jnp.maximum(dx[...] @ w1[...] + b1[...], 0.0)
    x = jnp.maximum(x @ w2[...] + b2[...], 0.0)
    x = x @ w3[...] + b3[...]
    ev1, ev2, ev3 = e1[...], e2[...], e3[...]
    h = x @ t1x[...] + tb1[...]
    t1i_ = t1i[...]
    pairs = ((x, ev1), (x, ev2), (x, ev3), (ev1, ev2), (ev1, ev3), (ev2, ev3))
    for k, (a, b) in enumerate(pairs):
        d = jnp.sum(a * b, axis=1, keepdims=True)
        h = h + d * t1i_[k:k + 1, :]
    h = jnp.maximum(h, 0.0)
    h = jnp.maximum(h @ t2[...] + tb2[...], 0.0)
    o = jnp.sum(h * t3r[...], axis=1, keepdims=True) + tb3[...]
    out[...] = jax.nn.sigmoid(o)


def _tc_call(dx, e1, e2, e3, w1, b1, w2, b2, w3, b3,
             t1x, t1i, tb1, t2, tb2, t3r, tb3, *, bs, interpret=False):
    B, DENSE = dx.shape
    D = e1.shape[1]
    grid = (B // bs,)
    row = lambda i: (i, 0)
    rep = lambda i: (0, 0)

    def spec(shape, imap):
        return pl.BlockSpec(shape, imap)

    return pl.pallas_call(
        _tc_body,
        grid=grid,
        in_specs=[
            spec((bs, DENSE), row),
            spec((bs, D), row), spec((bs, D), row), spec((bs, D), row),
            spec(w1.shape, rep), spec(b1.shape, rep),
            spec(w2.shape, rep), spec(b2.shape, rep),
            spec(w3.shape, rep), spec(b3.shape, rep),
            spec(t1x.shape, rep), spec(t1i.shape, rep), spec(tb1.shape, rep),
            spec(t2.shape, rep), spec(tb2.shape, rep),
            spec(t3r.shape, rep), spec(tb3.shape, rep),
        ],
        out_specs=spec((bs, 1), row),
        out_shape=jax.ShapeDtypeStruct((B, 1), jnp.float32),
        compiler_params=pltpu.CompilerParams(
            dimension_semantics=("parallel",)),
        interpret=interpret,
    )(dx, e1, e2, e3, w1, b1, w2, b2, w3, b3,
      t1x, t1i, tb1, t2, tb2, t3r, tb3)


def kernel(dense_x, idx1, off1, idx2, off2, idx3, off3,
           W1, b1, W2, b2, W3, b3, E1, E2, E3, T1, tb1, T2, tb2, T3, tb3):
    B, _ = dense_x.shape
    D = E1.shape[1]
    i1 = idx1.astype(jnp.int32)
    i2 = idx2.astype(jnp.int32)
    i3 = idx3.astype(jnp.int32)

    gather3 = _make_gather3(B, D)
    e1, e2, e3 = gather3(E1, i1, E2, i2, E3, i3)

    t1x = T1[:D]
    t1i = T1[D:]
    out = _tc_call(
        dense_x, e1, e2, e3,
        W1, b1.reshape(1, -1), W2, b2.reshape(1, -1), W3, b3.reshape(1, -1),
        t1x, t1i, tb1.reshape(1, -1), T2, tb2.reshape(1, -1),
        T3.reshape(1, -1), tb3.reshape(1, 1), bs=2048)
    return out


# speedup vs baseline: 1.5637x; 1.5637x over previous
import functools

import jax
import jax.numpy as jnp
from jax import lax
from jax.experimental import pallas as pl
from jax.experimental.pallas import tpu as pltpu
from jax.experimental.pallas import tpu_sc as plsc


def _make_gather3(B, D):
    info = plsc.get_sparse_core_info()
    NC, NS = info.num_cores, info.num_subcores
    NW = NC * NS
    assert B % (8 * NW) == 0
    bpw = B // NW
    mesh = plsc.VectorSubcoreMesh(core_axis_name="c", subcore_axis_name="s")

    @functools.partial(
        pl.kernel,
        out_type=[jax.ShapeDtypeStruct((B, D), jnp.float32)] * 3,
        mesh=mesh,
        scratch_types=[
            pltpu.VMEM((bpw,), jnp.int32),
            pltpu.VMEM((bpw, D), jnp.float32),
            pltpu.VMEM((bpw,), jnp.int32),
            pltpu.VMEM((bpw, D), jnp.float32),
            pltpu.VMEM((bpw,), jnp.int32),
            pltpu.VMEM((bpw, D), jnp.float32),
            pltpu.SemaphoreType.DMA,
            pltpu.SemaphoreType.DMA,
            pltpu.SemaphoreType.DMA,
        ],
        compiler_params=pltpu.CompilerParams(use_tc_tiling_on_sc=False),
    )
    def gather3(e1h, i1h, e2h, i2h, e3h, i3h, o1, o2, o3,
                iv1, rv1, iv2, rv2, iv3, rv3, s1, s2, s3):
        wid = lax.axis_index("s") * NC + lax.axis_index("c")
        base = wid * bpw
        pltpu.sync_copy(i1h.at[pl.ds(base, bpw)], iv1)
        pltpu.sync_copy(i2h.at[pl.ds(base, bpw)], iv2)
        pltpu.sync_copy(i3h.at[pl.ds(base, bpw)], iv3)
        c1 = pltpu.async_copy(e1h.at[iv1], rv1, s1)
        c2 = pltpu.async_copy(e2h.at[iv2], rv2, s2)
        c3 = pltpu.async_copy(e3h.at[iv3], rv3, s3)
        c1.wait()
        pltpu.sync_copy(rv1, o1.at[pl.ds(base, bpw)])
        c2.wait()
        pltpu.sync_copy(rv2, o2.at[pl.ds(base, bpw)])
        c3.wait()
        pltpu.sync_copy(rv3, o3.at[pl.ds(base, bpw)])

    return gather3


def _tc_body(dx, e1, e2, e3, w1, b1, w2, b2, w3, b3,
             t1x, t1i, tb1, t2, tb2, t3r, tb3, out):
    x = jnp.maximum(dx[...] @ w1[...] + b1[...], 0.0)
    x = jnp.maximum(x @ w2[...] + b2[...], 0.0)
    x = x @ w3[...] + b3[...]
    ev1, ev2, ev3 = e1[...], e2[...], e3[...]
    h = x @ t1x[...] + tb1[...]
    t1i_ = t1i[...]
    pairs = ((x, ev1), (x, ev2), (x, ev3), (ev1, ev2), (ev1, ev3), (ev2, ev3))
    for k, (a, b) in enumerate(pairs):
        d = jnp.sum(a * b, axis=1, keepdims=True)
        h = h + d * t1i_[k:k + 1, :]
    h = jnp.maximum(h, 0.0)
    h = jnp.maximum(h @ t2[...] + tb2[...], 0.0)
    o = jnp.sum(h * t3r[...], axis=1, keepdims=True) + tb3[...]
    out[...] = jax.nn.sigmoid(o)


def _tc_call(dx, e1, e2, e3, w1, b1, w2, b2, w3, b3,
             t1x, t1i, tb1, t2, tb2, t3r, tb3, *, bs, interpret=False):
    B, DENSE = dx.shape
    D = e1.shape[1]
    grid = (B // bs,)
    row = lambda i: (i, 0)
    rep = lambda i: (0, 0)

    def spec(shape, imap):
        return pl.BlockSpec(shape, imap)

    return pl.pallas_call(
        _tc_body,
        grid=grid,
        in_specs=[
            spec((bs, DENSE), row),
            spec((bs, D), row), spec((bs, D), row), spec((bs, D), row),
            spec(w1.shape, rep), spec(b1.shape, rep),
            spec(w2.shape, rep), spec(b2.shape, rep),
            spec(w3.shape, rep), spec(b3.shape, rep),
            spec(t1x.shape, rep), spec(t1i.shape, rep), spec(tb1.shape, rep),
            spec(t2.shape, rep), spec(tb2.shape, rep),
            spec(t3r.shape, rep), spec(tb3.shape, rep),
        ],
        out_specs=spec((bs, 1), row),
        out_shape=jax.ShapeDtypeStruct((B, 1), jnp.float32),
        compiler_params=pltpu.CompilerParams(
            dimension_semantics=("parallel",)),
        interpret=interpret,
    )(dx, e1, e2, e3, w1, b1, w2, b2, w3, b3,
      t1x, t1i, tb1, t2, tb2, t3r, tb3)


def kernel(dense_x, idx1, off1, idx2, off2, idx3, off3,
           W1, b1, W2, b2, W3, b3, E1, E2, E3, T1, tb1, T2, tb2, T3, tb3):
    B, _ = dense_x.shape
    D = E1.shape[1]
    i1 = idx1.astype(jnp.int32)
    i2 = idx2.astype(jnp.int32)
    i3 = idx3.astype(jnp.int32)

    gather3 = _make_gather3(B, D)
    e1, e2, e3 = gather3(E1, i1, E2, i2, E3, i3)

    t1x = T1[:D]
    t1i = T1[D:]
    out = _tc_call(
        dense_x, e1, e2, e3,
        W1, b1.reshape(1, -1), W2, b2.reshape(1, -1), W3, b3.reshape(1, -1),
        t1x, t1i, tb1.reshape(1, -1), T2, tb2.reshape(1, -1),
        T3.reshape(1, -1), tb3.reshape(1, 1), bs=2048)
    return out


# speedup vs baseline: 1.6637x; 1.0639x over previous
import functools

import jax
import jax.numpy as jnp
from jax import lax
from jax.experimental import pallas as pl
from jax.experimental.pallas import tpu as pltpu
from jax.experimental.pallas import tpu_sc as plsc


def _make_gather3(B, DP):
    info = plsc.get_sparse_core_info()
    NC, NS = info.num_cores, info.num_subcores
    NW = NC * NS
    assert B % (8 * NW) == 0
    bpw = B // NW
    mesh = plsc.VectorSubcoreMesh(core_axis_name="c", subcore_axis_name="s")

    @functools.partial(
        pl.kernel,
        out_type=[jax.ShapeDtypeStruct((B, DP), jnp.float32)] * 3,
        mesh=mesh,
        scratch_types=[
            pltpu.VMEM((bpw,), jnp.int32),
            pltpu.VMEM((bpw,), jnp.int32),
            pltpu.VMEM((bpw,), jnp.int32),
            pltpu.VMEM((bpw, DP), jnp.float32),
            pltpu.SemaphoreType.DMA,
        ],
    )
    def gather3(e1h, i1h, e2h, i2h, e3h, i3h, o1, o2, o3,
                iv1, iv2, iv3, rv, sem):
        wid = lax.axis_index("s") * NC + lax.axis_index("c")
        base = wid * bpw
        pltpu.sync_copy(i1h.at[pl.ds(base, bpw)], iv1)
        pltpu.sync_copy(i2h.at[pl.ds(base, bpw)], iv2)
        pltpu.sync_copy(i3h.at[pl.ds(base, bpw)], iv3)
        pltpu.async_copy(e1h.at[iv1], rv, sem).wait()
        pltpu.sync_copy(rv, o1.at[pl.ds(base, bpw)])
        pltpu.async_copy(e2h.at[iv2], rv, sem).wait()
        pltpu.sync_copy(rv, o2.at[pl.ds(base, bpw)])
        pltpu.async_copy(e3h.at[iv3], rv, sem).wait()
        pltpu.sync_copy(rv, o3.at[pl.ds(base, bpw)])

    return gather3


def _tc_body(dx, e1, e2, e3, w1, b1, w2, b2, w3, b3,
             t1x, t1i, tb1, t2, tb2, t3r, tb3, out):
    x = jnp.maximum(dx[...] @ w1[...] + b1[...], 0.0)
    x = jnp.maximum(x @ w2[...] + b2[...], 0.0)
    x = x @ w3[...] + b3[...]
    D = x.shape[1]
    ev1 = e1[...][:, :D]
    ev2 = e2[...][:, :D]
    ev3 = e3[...][:, :D]
    h = x @ t1x[...] + tb1[...]
    t1i_ = t1i[...]
    pairs = ((x, ev1), (x, ev2), (x, ev3), (ev1, ev2), (ev1, ev3), (ev2, ev3))
    for k, (a, b) in enumerate(pairs):
        d = jnp.sum(a * b, axis=1, keepdims=True)
        h = h + d * t1i_[k:k + 1, :]
    h = jnp.maximum(h, 0.0)
    h = jnp.maximum(h @ t2[...] + tb2[...], 0.0)
    o = jnp.sum(h * t3r[...], axis=1, keepdims=True) + tb3[...]
    out[...] = jax.nn.sigmoid(o)


def _tc_call(dx, e1, e2, e3, w1, b1, w2, b2, w3, b3,
             t1x, t1i, tb1, t2, tb2, t3r, tb3, *, bs, interpret=False):
    B, DENSE = dx.shape
    DP = e1.shape[1]
    grid = (B // bs,)
    row = lambda i: (i, 0)
    rep = lambda i: (0, 0)

    def spec(shape, imap):
        return pl.BlockSpec(shape, imap)

    return pl.pallas_call(
        _tc_body,
        grid=grid,
        in_specs=[
            spec((bs, DENSE), row),
            spec((bs, DP), row), spec((bs, DP), row), spec((bs, DP), row),
            spec(w1.shape, rep), spec(b1.shape, rep),
            spec(w2.shape, rep), spec(b2.shape, rep),
            spec(w3.shape, rep), spec(b3.shape, rep),
            spec(t1x.shape, rep), spec(t1i.shape, rep), spec(tb1.shape, rep),
            spec(t2.shape, rep), spec(tb2.shape, rep),
            spec(t3r.shape, rep), spec(tb3.shape, rep),
        ],
        out_specs=spec((bs, 1), row),
        out_shape=jax.ShapeDtypeStruct((B, 1), jnp.float32),
        compiler_params=pltpu.CompilerParams(
            dimension_semantics=("parallel",)),
        interpret=interpret,
    )(dx, e1, e2, e3, w1, b1, w2, b2, w3, b3,
      t1x, t1i, tb1, t2, tb2, t3r, tb3)


def kernel(dense_x, idx1, off1, idx2, off2, idx3, off3,
           W1, b1, W2, b2, W3, b3, E1, E2, E3, T1, tb1, T2, tb2, T3, tb3):
    B, _ = dense_x.shape
    D = E1.shape[1]
    DP = 128
    i1 = idx1.astype(jnp.int32)
    i2 = idx2.astype(jnp.int32)
    i3 = idx3.astype(jnp.int32)

    pad = lambda E: jnp.pad(E, ((0, 0), (0, DP - D)))
    gather3 = _make_gather3(B, DP)
    e1, e2, e3 = gather3(pad(E1), i1, pad(E2), i2, pad(E3), i3)

    t1x = T1[:D]
    t1i = T1[D:]
    out = _tc_call(
        dense_x, e1, e2, e3,
        W1, b1.reshape(1, -1), W2, b2.reshape(1, -1), W3, b3.reshape(1, -1),
        t1x, t1i, tb1.reshape(1, -1), T2, tb2.reshape(1, -1),
        T3.reshape(1, -1), tb3.reshape(1, 1), bs=2048)
    return out


# speedup vs baseline: 2.4882x; 1.4956x over previous
import functools
import math

import jax
import jax.numpy as jnp
from jax import lax
from jax.experimental import pallas as pl
from jax.experimental.pallas import tpu as pltpu
from jax.experimental.pallas import tpu_sc as plsc

DP = 128


def _tp_body(xr, outr):
    xt = jnp.transpose(xr[...])
    outr[...] = jnp.concatenate([xt, xt], axis=1)


def _stage_rows(ET, *, W=8192):
    D, V = ET.shape
    grid = (math.ceil(V / W),)
    return pl.pallas_call(
        _tp_body,
        grid=grid,
        in_specs=[pl.BlockSpec((D, W), lambda i: (0, i))],
        out_specs=pl.BlockSpec((W, DP), lambda i: (i, 0)),
        out_shape=jax.ShapeDtypeStruct((V, DP), jnp.float32),
        compiler_params=pltpu.CompilerParams(
            dimension_semantics=("arbitrary",)),
    )(ET)


def _make_gather3(B):
    info = plsc.get_sparse_core_info()
    NC, NS = info.num_cores, info.num_subcores
    NW = NC * NS
    assert B % (8 * NW) == 0
    bpw = B // NW
    mesh = plsc.VectorSubcoreMesh(core_axis_name="c", subcore_axis_name="s")

    @functools.partial(
        pl.kernel,
        out_type=[jax.ShapeDtypeStruct((B, DP), jnp.float32)] * 3,
        mesh=mesh,
        scratch_types=[
            pltpu.VMEM((bpw,), jnp.int32),
            pltpu.VMEM((bpw,), jnp.int32),
            pltpu.VMEM((bpw,), jnp.int32),
            pltpu.VMEM((bpw, DP), jnp.float32),
            pltpu.SemaphoreType.DMA,
        ],
    )
    def gather3(e1h, i1h, e2h, i2h, e3h, i3h, o1, o2, o3,
                iv1, iv2, iv3, rv, sem):
        wid = lax.axis_index("s") * NC + lax.axis_index("c")
        base = wid * bpw
        pltpu.sync_copy(i1h.at[pl.ds(base, bpw)], iv1)
        pltpu.sync_copy(i2h.at[pl.ds(base, bpw)], iv2)
        pltpu.sync_copy(i3h.at[pl.ds(base, bpw)], iv3)
        pltpu.async_copy(e1h.at[iv1], rv, sem).wait()
        pltpu.sync_copy(rv, o1.at[pl.ds(base, bpw)])
        pltpu.async_copy(e2h.at[iv2], rv, sem).wait()
        pltpu.sync_copy(rv, o2.at[pl.ds(base, bpw)])
        pltpu.async_copy(e3h.at[iv3], rv, sem).wait()
        pltpu.sync_copy(rv, o3.at[pl.ds(base, bpw)])

    return gather3


def _tc_body(dx, e1, e2, e3, w1, b1, w2, b2, w3, b3,
             t1x, t1i, tb1, t2, tb2, t3r, tb3, out):
    x = jnp.maximum(dx[...] @ w1[...] + b1[...], 0.0)
    x = jnp.maximum(x @ w2[...] + b2[...], 0.0)
    x = x @ w3[...] + b3[...]
    D = x.shape[1]
    ev1 = e1[...][:, :D]
    ev2 = e2[...][:, :D]
    ev3 = e3[...][:, :D]
    h = x @ t1x[...] + tb1[...]
    t1i_ = t1i[...]
    pairs = ((x, ev1), (x, ev2), (x, ev3), (ev1, ev2), (ev1, ev3), (ev2, ev3))
    for k, (a, b) in enumerate(pairs):
        d = jnp.sum(a * b, axis=1, keepdims=True)
        h = h + d * t1i_[k:k + 1, :]
    h = jnp.maximum(h, 0.0)
    h = jnp.maximum(h @ t2[...] + tb2[...], 0.0)
    o = jnp.sum(h * t3r[...], axis=1, keepdims=True) + tb3[...]
    out[...] = jax.nn.sigmoid(o)


def _tc_call(dx, e1, e2, e3, w1, b1, w2, b2, w3, b3,
             t1x, t1i, tb1, t2, tb2, t3r, tb3, *, bs, interpret=False):
    B, DENSE = dx.shape
    grid = (B // bs,)
    row = lambda i: (i, 0)
    rep = lambda i: (0, 0)

    def spec(shape, imap):
        return pl.BlockSpec(shape, imap)

    return pl.pallas_call(
        _tc_body,
        grid=grid,
        in_specs=[
            spec((bs, DENSE), row),
            spec((bs, DP), row), spec((bs, DP), row), spec((bs, DP), row),
            spec(w1.shape, rep), spec(b1.shape, rep),
            spec(w2.shape, rep), spec(b2.shape, rep),
            spec(w3.shape, rep), spec(b3.shape, rep),
            spec(t1x.shape, rep), spec(t1i.shape, rep), spec(tb1.shape, rep),
            spec(t2.shape, rep), spec(tb2.shape, rep),
            spec(t3r.shape, rep), spec(tb3.shape, rep),
        ],
        out_specs=spec((bs, 1), row),
        out_shape=jax.ShapeDtypeStruct((B, 1), jnp.float32),
        compiler_params=pltpu.CompilerParams(
            dimension_semantics=("parallel",)),
        interpret=interpret,
    )(dx, e1, e2, e3, w1, b1, w2, b2, w3, b3,
      t1x, t1i, tb1, t2, tb2, t3r, tb3)


def kernel(dense_x, idx1, off1, idx2, off2, idx3, off3,
           W1, b1, W2, b2, W3, b3, E1, E2, E3, T1, tb1, T2, tb2, T3, tb3):
    B, _ = dense_x.shape
    D = E1.shape[1]
    i1 = idx1.astype(jnp.int32)
    i2 = idx2.astype(jnp.int32)
    i3 = idx3.astype(jnp.int32)

    s1 = _stage_rows(E1.T)
    s2 = _stage_rows(E2.T)
    s3 = _stage_rows(E3.T)
    gather3 = _make_gather3(B)
    e1, e2, e3 = gather3(s1, i1, s2, i2, s3, i3)

    t1x = T1[:D]
    t1i = T1[D:]
    out = _tc_call(
        dense_x, e1, e2, e3,
        W1, b1.reshape(1, -1), W2, b2.reshape(1, -1), W3, b3.reshape(1, -1),
        t1x, t1i, tb1.reshape(1, -1), T2, tb2.reshape(1, -1),
        T3.reshape(1, -1), tb3.reshape(1, 1), bs=2048)
    return out


# speedup vs baseline: 3.0417x; 1.2224x over previous
import functools
import math

import jax
import jax.numpy as jnp
from jax import lax
from jax.experimental import pallas as pl
from jax.experimental.pallas import tpu as pltpu
from jax.experimental.pallas import tpu_sc as plsc

DP = 128
LW = 13
W = 1 << LW


def _tp_body(x0r, x1r, outr):
    xt0 = jnp.transpose(x0r[...])
    xt1 = jnp.transpose(x1r[...])
    outr[...] = jnp.concatenate([xt0, xt1], axis=1)


def _stage_rows(ET):
    D, V = ET.shape
    grid = (math.ceil(V / (2 * W)),)
    nrows = grid[0] * W
    maxb = math.ceil(V / W) - 1
    return pl.pallas_call(
        _tp_body,
        grid=grid,
        in_specs=[
            pl.BlockSpec((D, W), lambda i: (0, jnp.minimum(2 * i, maxb))),
            pl.BlockSpec((D, W), lambda i: (0, jnp.minimum(2 * i + 1, maxb))),
        ],
        out_specs=pl.BlockSpec((W, DP), lambda i: (i, 0)),
        out_shape=jax.ShapeDtypeStruct((nrows, DP), jnp.float32),
        compiler_params=pltpu.CompilerParams(
            dimension_semantics=("arbitrary",)),
    )(ET, ET)


def _make_gather3(B):
    info = plsc.get_sparse_core_info()
    NC, NS = info.num_cores, info.num_subcores
    NW = NC * NS
    assert B % (8 * NW) == 0
    bpw = B // NW
    mesh = plsc.VectorSubcoreMesh(core_axis_name="c", subcore_axis_name="s")

    @functools.partial(
        pl.kernel,
        out_type=[jax.ShapeDtypeStruct((B, DP), jnp.float32)] * 3,
        mesh=mesh,
        scratch_types=[
            pltpu.VMEM((bpw,), jnp.int32),
            pltpu.VMEM((bpw,), jnp.int32),
            pltpu.VMEM((bpw,), jnp.int32),
            pltpu.VMEM((bpw, DP), jnp.float32),
            pltpu.SemaphoreType.DMA,
        ],
    )
    def gather3(e1h, i1h, e2h, i2h, e3h, i3h, o1, o2, o3,
                iv1, iv2, iv3, rv, sem):
        wid = lax.axis_index("s") * NC + lax.axis_index("c")
        base = wid * bpw
        pltpu.sync_copy(i1h.at[pl.ds(base, bpw)], iv1)
        pltpu.sync_copy(i2h.at[pl.ds(base, bpw)], iv2)
        pltpu.sync_copy(i3h.at[pl.ds(base, bpw)], iv3)
        for iv in (iv1, iv2, iv3):
            for t in range(bpw // 16):
                x = iv[pl.ds(t * 16, 16)]
                hi = lax.shift_left(
                    lax.shift_right_logical(x, LW + 1), LW)
                lo = jnp.bitwise_and(x, W - 1)
                iv[pl.ds(t * 16, 16)] = jnp.bitwise_or(hi, lo)
        pltpu.async_copy(e1h.at[iv1], rv, sem).wait()
        pltpu.sync_copy(rv, o1.at[pl.ds(base, bpw)])
        pltpu.async_copy(e2h.at[iv2], rv, sem).wait()
        pltpu.sync_copy(rv, o2.at[pl.ds(base, bpw)])
        pltpu.async_copy(e3h.at[iv3], rv, sem).wait()
        pltpu.sync_copy(rv, o3.at[pl.ds(base, bpw)])

    return gather3


def _tc_body(dx, e1, e2, e3, p1, p2, p3, w1, b1, w2, b2, w3, b3,
             t1x, t1i, tb1, t2, tb2, t3r, tb3, out):
    x = jnp.maximum(dx[...] @ w1[...] + b1[...], 0.0)
    x = jnp.maximum(x @ w2[...] + b2[...], 0.0)
    x = x @ w3[...] + b3[...]
    D = x.shape[1]

    def half(e, p):
        ep = e[...]
        return jnp.where(p[...] > 0, ep[:, D:], ep[:, :D])

    ev1 = half(e1, p1)
    ev2 = half(e2, p2)
    ev3 = half(e3, p3)
    h = x @ t1x[...] + tb1[...]
    t1i_ = t1i[...]
    pairs = ((x, ev1), (x, ev2), (x, ev3), (ev1, ev2), (ev1, ev3), (ev2, ev3))
    for k, (a, b) in enumerate(pairs):
        d = jnp.sum(a * b, axis=1, keepdims=True)
        h = h + d * t1i_[k:k + 1, :]
    h = jnp.maximum(h, 0.0)
    h = jnp.maximum(h @ t2[...] + tb2[...], 0.0)
    o = jnp.sum(h * t3r[...], axis=1, keepdims=True) + tb3[...]
    out[...] = jax.nn.sigmoid(o)


def _tc_call(dx, e1, e2, e3, p1, p2, p3, w1, b1, w2, b2, w3, b3,
             t1x, t1i, tb1, t2, tb2, t3r, tb3, *, bs, interpret=False):
    B, DENSE = dx.shape
    grid = (B // bs,)
    row = lambda i: (i, 0)
    rep = lambda i: (0, 0)

    def spec(shape, imap):
        return pl.BlockSpec(shape, imap)

    return pl.pallas_call(
        _tc_body,
        grid=grid,
        in_specs=[
            spec((bs, DENSE), row),
            spec((bs, DP), row), spec((bs, DP), row), spec((bs, DP), row),
            spec((bs, 1), row), spec((bs, 1), row), spec((bs, 1), row),
            spec(w1.shape, rep), spec(b1.shape, rep),
            spec(w2.shape, rep), spec(b2.shape, rep),
            spec(w3.shape, rep), spec(b3.shape, rep),
            spec(t1x.shape, rep), spec(t1i.shape, rep), spec(tb1.shape, rep),
            spec(t2.shape, rep), spec(tb2.shape, rep),
            spec(t3r.shape, rep), spec(tb3.shape, rep),
        ],
        out_specs=spec((bs, 1), row),
        out_shape=jax.ShapeDtypeStruct((B, 1), jnp.float32),
        compiler_params=pltpu.CompilerParams(
            dimension_semantics=("parallel",)),
        interpret=interpret,
    )(dx, e1, e2, e3, p1, p2, p3, w1, b1, w2, b2, w3, b3,
      t1x, t1i, tb1, t2, tb2, t3r, tb3)


def kernel(dense_x, idx1, off1, idx2, off2, idx3, off3,
           W1, b1, W2, b2, W3, b3, E1, E2, E3, T1, tb1, T2, tb2, T3, tb3):
    B, _ = dense_x.shape
    D = E1.shape[1]
    i1 = idx1.astype(jnp.int32)
    i2 = idx2.astype(jnp.int32)
    i3 = idx3.astype(jnp.int32)

    s1 = _stage_rows(E1.T)
    s2 = _stage_rows(E2.T)
    s3 = _stage_rows(E3.T)
    gather3 = _make_gather3(B)
    e1, e2, e3 = gather3(s1, i1, s2, i2, s3, i3)
    p1 = ((i1 >> LW) & 1).reshape(B, 1)
    p2 = ((i2 >> LW) & 1).reshape(B, 1)
    p3 = ((i3 >> LW) & 1).reshape(B, 1)

    t1x = T1[:D]
    t1i = T1[D:]
    out = _tc_call(
        dense_x, e1, e2, e3, p1, p2, p3,
        W1, b1.reshape(1, -1), W2, b2.reshape(1, -1), W3, b3.reshape(1, -1),
        t1x, t1i, tb1.reshape(1, -1), T2, tb2.reshape(1, -1),
        T3.reshape(1, -1), tb3.reshape(1, 1), bs=2048)
    return out


# speedup vs baseline: 3.1576x; 1.0381x over previous
import functools
import math

import jax
import jax.numpy as jnp
from jax import lax
from jax.experimental import pallas as pl
from jax.experimental.pallas import tpu as pltpu
from jax.experimental.pallas import tpu_sc as plsc

DP = 128
LW = 14
W = 1 << LW


def _tp_body(x0r, x1r, outr):
    xt0 = jnp.transpose(x0r[...])
    xt1 = jnp.transpose(x1r[...])
    outr[...] = jnp.concatenate([xt0, xt1], axis=1)


def _stage_rows(ET):
    D, V = ET.shape
    grid = (math.ceil(V / (2 * W)),)
    nrows = grid[0] * W
    maxb = math.ceil(V / W) - 1
    return pl.pallas_call(
        _tp_body,
        grid=grid,
        in_specs=[
            pl.BlockSpec((D, W), lambda i: (0, jnp.minimum(2 * i, maxb))),
            pl.BlockSpec((D, W), lambda i: (0, jnp.minimum(2 * i + 1, maxb))),
        ],
        out_specs=pl.BlockSpec((W, DP), lambda i: (i, 0)),
        out_shape=jax.ShapeDtypeStruct((nrows, DP), jnp.float32),
        compiler_params=pltpu.CompilerParams(
            dimension_semantics=("arbitrary",)),
    )(ET, ET)


def _make_gather3(B):
    info = plsc.get_sparse_core_info()
    NC, NS = info.num_cores, info.num_subcores
    NW = NC * NS
    assert B % (8 * NW) == 0
    bpw = B // NW
    mesh = plsc.VectorSubcoreMesh(core_axis_name="c", subcore_axis_name="s")

    @functools.partial(
        pl.kernel,
        out_type=[jax.ShapeDtypeStruct((B, DP), jnp.float32)] * 3,
        mesh=mesh,
        scratch_types=[
            pltpu.VMEM((bpw,), jnp.int32),
            pltpu.VMEM((bpw,), jnp.int32),
            pltpu.VMEM((bpw,), jnp.int32),
            pltpu.VMEM((bpw, DP), jnp.float32),
            pltpu.SemaphoreType.DMA,
        ],
    )
    def gather3(e1h, i1h, e2h, i2h, e3h, i3h, o1, o2, o3,
                iv1, iv2, iv3, rv, sem):
        wid = lax.axis_index("s") * NC + lax.axis_index("c")
        base = wid * bpw
        pltpu.sync_copy(i1h.at[pl.ds(base, bpw)], iv1)
        pltpu.sync_copy(i2h.at[pl.ds(base, bpw)], iv2)
        pltpu.sync_copy(i3h.at[pl.ds(base, bpw)], iv3)
        for iv in (iv1, iv2, iv3):
            for t in range(bpw // 16):
                x = iv[pl.ds(t * 16, 16)]
                hi = lax.shift_left(
                    lax.shift_right_logical(x, LW + 1), LW)
                lo = jnp.bitwise_and(x, W - 1)
                iv[pl.ds(t * 16, 16)] = jnp.bitwise_or(hi, lo)
        pltpu.async_copy(e1h.at[iv1], rv, sem).wait()
        pltpu.sync_copy(rv, o1.at[pl.ds(base, bpw)])
        pltpu.async_copy(e2h.at[iv2], rv, sem).wait()
        pltpu.sync_copy(rv, o2.at[pl.ds(base, bpw)])
        pltpu.async_copy(e3h.at[iv3], rv, sem).wait()
        pltpu.sync_copy(rv, o3.at[pl.ds(base, bpw)])

    return gather3


def _tc_body(dx, e1, e2, e3, p1, p2, p3, w1, b1, w2, b2, w3, b3,
             t1x, t1i, tb1, t2, tb2, t3r, tb3, out):
    x = jnp.maximum(dx[...] @ w1[...] + b1[...], 0.0)
    x = jnp.maximum(x @ w2[...] + b2[...], 0.0)
    x = x @ w3[...] + b3[...]
    D = x.shape[1]

    def half(e, p):
        ep = e[...]
        return jnp.where(p[...] > 0, ep[:, D:], ep[:, :D])

    ev1 = half(e1, p1)
    ev2 = half(e2, p2)
    ev3 = half(e3, p3)
    h = x @ t1x[...] + tb1[...]
    t1i_ = t1i[...]
    pairs = ((x, ev1), (x, ev2), (x, ev3), (ev1, ev2), (ev1, ev3), (ev2, ev3))
    for k, (a, b) in enumerate(pairs):
        d = jnp.sum(a * b, axis=1, keepdims=True)
        h = h + d * t1i_[k:k + 1, :]
    h = jnp.maximum(h, 0.0)
    h = jnp.maximum(h @ t2[...] + tb2[...], 0.0)
    o = jnp.sum(h * t3r[...], axis=1, keepdims=True) + tb3[...]
    out[...] = jax.nn.sigmoid(o)


def _tc_call(dx, e1, e2, e3, p1, p2, p3, w1, b1, w2, b2, w3, b3,
             t1x, t1i, tb1, t2, tb2, t3r, tb3, *, bs, interpret=False):
    B, DENSE = dx.shape
    grid = (B // bs,)
    row = lambda i: (i, 0)
    rep = lambda i: (0, 0)

    def spec(shape, imap):
        return pl.BlockSpec(shape, imap)

    return pl.pallas_call(
        _tc_body,
        grid=grid,
        in_specs=[
            spec((bs, DENSE), row),
            spec((bs, DP), row), spec((bs, DP), row), spec((bs, DP), row),
            spec((bs, 1), row), spec((bs, 1), row), spec((bs, 1), row),
            spec(w1.shape, rep), spec(b1.shape, rep),
            spec(w2.shape, rep), spec(b2.shape, rep),
            spec(w3.shape, rep), spec(b3.shape, rep),
            spec(t1x.shape, rep), spec(t1i.shape, rep), spec(tb1.shape, rep),
            spec(t2.shape, rep), spec(tb2.shape, rep),
            spec(t3r.shape, rep), spec(tb3.shape, rep),
        ],
        out_specs=spec((bs, 1), row),
        out_shape=jax.ShapeDtypeStruct((B, 1), jnp.float32),
        compiler_params=pltpu.CompilerParams(
            dimension_semantics=("parallel",)),
        interpret=interpret,
    )(dx, e1, e2, e3, p1, p2, p3, w1, b1, w2, b2, w3, b3,
      t1x, t1i, tb1, t2, tb2, t3r, tb3)


def kernel(dense_x, idx1, off1, idx2, off2, idx3, off3,
           W1, b1, W2, b2, W3, b3, E1, E2, E3, T1, tb1, T2, tb2, T3, tb3):
    B, _ = dense_x.shape
    D = E1.shape[1]
    i1 = idx1.astype(jnp.int32)
    i2 = idx2.astype(jnp.int32)
    i3 = idx3.astype(jnp.int32)

    s1 = _stage_rows(E1.T)
    s2 = _stage_rows(E2.T)
    s3 = _stage_rows(E3.T)
    gather3 = _make_gather3(B)
    e1, e2, e3 = gather3(s1, i1, s2, i2, s3, i3)
    p1 = ((i1 >> LW) & 1).reshape(B, 1)
    p2 = ((i2 >> LW) & 1).reshape(B, 1)
    p3 = ((i3 >> LW) & 1).reshape(B, 1)

    t1x = T1[:D]
    t1i = T1[D:]
    out = _tc_call(
        dense_x, e1, e2, e3, p1, p2, p3,
        W1, b1.reshape(1, -1), W2, b2.reshape(1, -1), W3, b3.reshape(1, -1),
        t1x, t1i, tb1.reshape(1, -1), T2, tb2.reshape(1, -1),
        T3.reshape(1, -1), tb3.reshape(1, 1), bs=2048)
    return out
